# Initial kernel scaffold; baseline (speedup 1.0000x reference)
import jax
import jax.numpy as jnp
from jax.experimental import pallas as pl


def kernel(angles, table, W, b):
    raise NotImplementedError("write your pallas kernel here")


# speedup vs baseline: 3.5177x; 3.5177x over previous
import functools

import jax
import jax.numpy as jnp
from jax import lax
from jax.experimental import pallas as pl
from jax.experimental.pallas import tpu as pltpu
from jax.experimental.pallas import tpu_sc as plsc

D = 64
V = 360
BATCH = 16384
HIST = 200
BTOT = BATCH * HIST

NC = 2
NS = 16
NW = NC * NS
LANES = 128

ROWS_TOT = BTOT // LANES
ROWS_PER_W = ROWS_TOT // NW
K = 8
CHUNK = K * LANES
ITERS = ROWS_PER_W // K


def _table_transform_body(tbl_ref, w_ref, b_ref, out_ref):
    prod = lax.dot_general(
        tbl_ref[...], w_ref[...],
        (((1,), (1,)), ((), ())),
        preferred_element_type=jnp.float32,
    )
    out_ref[...] = jnp.maximum(prod + b_ref[...], 0.0)


def _table_transform(table, W, b2):
    return pl.pallas_call(
        _table_transform_body,
        out_shape=jax.ShapeDtypeStruct((V, D), jnp.float32),
    )(table, W, b2)


_MESH = plsc.VectorSubcoreMesh(core_axis_name="c", subcore_axis_name="s")


@functools.partial(
    pl.kernel,
    out_type=jax.ShapeDtypeStruct((BTOT, D), jnp.float32),
    mesh=_MESH,
    compiler_params=pltpu.CompilerParams(use_tc_tiling_on_sc=False),
    scratch_types=[
        pltpu.VMEM((K, LANES), jnp.int32),
        pltpu.VMEM((CHUNK, D), jnp.float32),
        pltpu.SemaphoreType.DMA,
    ],
)
def _gather_kernel(idx_hbm, t2_hbm, out_hbm, idx_v, rows_v, sem):
    wid = lax.axis_index("s") * NC + lax.axis_index("c")
    row0 = wid * ROWS_PER_W

    def step(i, carry):
        r = row0 + i * K
        pltpu.sync_copy(idx_hbm.at[pl.ds(r, K)], idx_v)
        descs = [
            pltpu.async_copy(
                t2_hbm.at[idx_v.at[j]],
                rows_v.at[pl.ds(j * LANES, LANES)],
                sem,
            )
            for j in range(K)
        ]
        for d_ in descs:
            d_.wait()
        pltpu.sync_copy(rows_v, out_hbm.at[pl.ds(r * LANES, CHUNK)])
        return carry

    lax.fori_loop(0, ITERS, step, 0)


def kernel(angles, table, W, b):
    t2 = _table_transform(table, W, b.reshape(1, D))
    idx = angles.astype(jnp.int32).reshape(ROWS_TOT, LANES)
    out = _gather_kernel(idx, t2)
    return out.reshape(BATCH, HIST, D)


# speedup vs baseline: 5.6676x; 1.6111x over previous
import functools

import jax
import jax.numpy as jnp
from jax import lax
from jax.experimental import pallas as pl
from jax.experimental.pallas import tpu as pltpu
from jax.experimental.pallas import tpu_sc as plsc

D = 64
V = 360
BATCH = 16384
HIST = 200
BTOT = BATCH * HIST

NC = 2
NS = 16
NW = NC * NS
LANES = 128

ROWS_TOT = BTOT // LANES
ROWS_PER_W = ROWS_TOT // NW
K = 4
CHUNK = K * LANES
ITERS = ROWS_PER_W // K
PAIRS = ITERS // 2


def _table_transform_body(tbl_ref, w_ref, b_ref, out_ref):
    prod = lax.dot_general(
        tbl_ref[...], w_ref[...],
        (((1,), (1,)), ((), ())),
        preferred_element_type=jnp.float32,
    )
    out_ref[...] = jnp.maximum(prod + b_ref[...], 0.0)


def _table_transform(table, W, b2):
    return pl.pallas_call(
        _table_transform_body,
        out_shape=jax.ShapeDtypeStruct((V, D), jnp.float32),
    )(table, W, b2)


_MESH = plsc.VectorSubcoreMesh(core_axis_name="c", subcore_axis_name="s")


@functools.partial(
    pl.kernel,
    out_type=jax.ShapeDtypeStruct((BTOT, D), jnp.float32),
    mesh=_MESH,
    compiler_params=pltpu.CompilerParams(use_tc_tiling_on_sc=False),
    scratch_types=[
        pltpu.VMEM_SHARED((V, D), jnp.float32),
        pltpu.VMEM((K, LANES), jnp.int32),
        pltpu.VMEM((K, LANES), jnp.int32),
        pltpu.VMEM((CHUNK, D), jnp.float32),
        pltpu.VMEM((CHUNK, D), jnp.float32),
        pltpu.SemaphoreType.DMA,
        pltpu.SemaphoreType.DMA,
        pltpu.SemaphoreType.DMA,
        pltpu.SemaphoreType.DMA,
    ],
)
def _gather_kernel(idx_hbm, t2_hbm, out_hbm, t2_sh,
                   idx_v0, idx_v1, rows_v0, rows_v1,
                   sem_idx, sem_g, sem_st0, sem_st1):
    cid = lax.axis_index("c")
    sid = lax.axis_index("s")
    wid = sid * NC + cid
    row0 = wid * ROWS_PER_W

    @pl.when(sid == 0)
    def _():
        pltpu.sync_copy(t2_hbm, t2_sh)

    plsc.subcore_barrier()

    idx_bufs = (idx_v0, idx_v1)
    rows_bufs = (rows_v0, rows_v1)
    st_sems = (sem_st0, sem_st1)

    pltpu.async_copy(idx_hbm.at[pl.ds(row0, K)], idx_v0, sem_idx)

    def pair(p, carry):
        for half in range(2):
            i = 2 * p + half
            r = row0 + i * K
            idx_v = idx_bufs[half]
            rows_v = rows_bufs[half]
            sem_st = st_sems[half]

            pltpu.make_async_copy(
                idx_hbm.at[pl.ds(r, K)], idx_v, sem_idx).wait()

            @pl.when(i + 1 < ITERS)
            def _():
                pltpu.async_copy(
                    idx_hbm.at[pl.ds(r + K, K)],
                    idx_bufs[1 - half], sem_idx)

            @pl.when(p > 0)
            def _():
                pltpu.make_async_copy(
                    rows_v, out_hbm.at[pl.ds(0, CHUNK)], sem_st).wait()

            descs = [
                pltpu.async_copy(
                    t2_sh.at[idx_v.at[j]],
                    rows_v.at[pl.ds(j * LANES, LANES)],
                    sem_g,
                )
                for j in range(K)
            ]
            for d_ in descs:
                d_.wait()

            pltpu.async_copy(
                rows_v, out_hbm.at[pl.ds(r * LANES, CHUNK)], sem_st)
        return carry

    lax.fori_loop(0, PAIRS, pair, 0)

    pltpu.make_async_copy(rows_v0, out_hbm.at[pl.ds(0, CHUNK)], sem_st0).wait()
    pltpu.make_async_copy(rows_v1, out_hbm.at[pl.ds(0, CHUNK)], sem_st1).wait()


def kernel(angles, table, W, b):
    t2 = _table_transform(table, W, b.reshape(1, D))
    idx = angles.astype(jnp.int32).reshape(ROWS_TOT, LANES)
    out = _gather_kernel(idx, t2)
    return out.reshape(BATCH, HIST, D)
